# Initial kernel scaffold; baseline (speedup 1.0000x reference)
import jax
import jax.numpy as jnp
from jax.experimental import pallas as pl


def kernel(query_up, key_up, lsh_proj_g0, lsh_proj_g1, head_idx):
    raise NotImplementedError("write your pallas kernel here")


# speedup vs baseline: 20.2362x; 20.2362x over previous
import functools

import jax
import jax.numpy as jnp
from jax.experimental import pallas as pl
from jax.experimental.pallas import tpu as pltpu

B = 2
L = 2048
D = 64
DG = 32
NH = 4
K = 64
BQ = 256
NEG = -1e9
THRESH = -1e8


def _body(q_ref, k_ref, p0t_ref, p1t_ref, os_ref, oi_ref, msk_ref, done_ref):
    qb = q_ref[0]
    kb = k_ref[0]
    p0t = p0t_ref[...]
    p1t = p1t_ref[...]

    dn = (((1,), (1,)), ((), ()))
    one = jnp.float32(1.0)
    sq = jnp.where(qb > 0, one, -one)
    sk = jnp.where(kb > 0, one, -one)
    st0 = jax.lax.dot_general(sq[:, :DG], sk[:, :DG], dn,
                              preferred_element_type=jnp.float32)
    st1 = jax.lax.dot_general(sq[:, DG:], sk[:, DG:], dn,
                              preferred_element_type=jnp.float32)
    trie0 = st0 > 31.5
    trie1 = st1 > 31.5

    def hq(x, pt):
        v = jax.lax.dot_general(x, pt, dn, preferred_element_type=jnp.float32)
        return jnp.floor(v * 0.5).astype(jnp.int32) & 63

    def hk(pt, x):
        v = jax.lax.dot_general(pt, x, dn, preferred_element_type=jnp.float32)
        return jnp.floor(v * 0.5).astype(jnp.int32) & 63

    qh0 = hq(qb[:, :DG], p0t)
    qh1 = hq(qb[:, DG:], p1t)
    kh0 = hk(p0t, kb[:, :DG])
    kh1 = hk(p1t, kb[:, DG:])

    lsh0 = qh0[:, 0:1] == kh0[0:1, :]
    lsh1 = qh1[:, 0:1] == kh1[0:1, :]
    for h in range(1, NH):
        lsh0 = lsh0 | (qh0[:, h:h + 1] == kh0[h:h + 1, :])
        lsh1 = lsh1 | (qh1[:, h:h + 1] == kh1[h:h + 1, :])

    mask = (trie0 & lsh0) | (trie1 & lsh1)
    npos = jnp.sum(mask.astype(jnp.int32))

    os_ref[0] = jnp.full((BQ, K), NEG, jnp.float32)
    oi_ref[0] = jnp.full((BQ, K), -1, jnp.int32)
    done_ref[0] = jnp.where(npos > 0, 0, 1).astype(jnp.int32)

    @pl.when(npos > 0)
    def _():
        scores = jax.lax.dot_general(qb, kb, dn,
                                     preferred_element_type=jnp.float32)
        msk_ref[...] = jnp.where(mask, scores, NEG)

    kiota = jax.lax.broadcasted_iota(jnp.int32, (BQ, L), 1)
    liota = jax.lax.broadcasted_iota(jnp.int32, (BQ, K), 1)

    def step(j, carry):
        @pl.when(done_ref[0] == 0)
        def _():
            mm = msk_ref[...]
            m = jnp.max(mm, axis=1)
            bmax = jnp.max(m)

            @pl.when(bmax <= THRESH)
            def _():
                done_ref[0] = 1

            @pl.when(bmax > THRESH)
            def _():
                eq = mm == m[:, None]
                am = jnp.min(jnp.where(eq, kiota, L), axis=1)
                valid = m > THRESH
                sc = jnp.where(valid, m, NEG)
                ix = jnp.where(valid, am, -1)
                os_ref[0] = jnp.where(liota == j, sc[:, None], os_ref[0])
                oi_ref[0] = jnp.where(liota == j, ix[:, None], oi_ref[0])
                pop = (kiota == am[:, None]) & valid[:, None]
                msk_ref[...] = jnp.where(pop, NEG, mm)
        return carry

    jax.lax.fori_loop(0, K, step, 0)


@jax.jit
def _run(q, k, p0t, p1t):
    out = pl.pallas_call(
        _body,
        grid=(B, L // BQ),
        in_specs=[
            pl.BlockSpec((1, BQ, D), lambda b, i: (b, i, 0)),
            pl.BlockSpec((1, L, D), lambda b, i: (b, 0, 0)),
            pl.BlockSpec((NH, DG), lambda b, i: (0, 0)),
            pl.BlockSpec((NH, DG), lambda b, i: (0, 0)),
        ],
        out_specs=[
            pl.BlockSpec((1, BQ, K), lambda b, i: (b, i, 0)),
            pl.BlockSpec((1, BQ, K), lambda b, i: (b, i, 0)),
        ],
        out_shape=[
            jax.ShapeDtypeStruct((B, L, K), jnp.float32),
            jax.ShapeDtypeStruct((B, L, K), jnp.int32),
        ],
        scratch_shapes=[
            pltpu.VMEM((BQ, L), jnp.float32),
            pltpu.SMEM((1,), jnp.int32),
        ],
    )(q, k, p0t, p1t)
    return out[0], out[1]


def kernel(query_up, key_up, lsh_proj_g0, lsh_proj_g1, head_idx=0):
    del head_idx
    return _run(query_up, key_up, lsh_proj_g0.T, lsh_proj_g1.T)


# speedup vs baseline: 23.1093x; 1.1420x over previous
import jax
import jax.numpy as jnp
from jax.experimental import pallas as pl
from jax.experimental.pallas import tpu as pltpu

B = 2
L = 2048
D = 64
DG = 32
NH = 4
NB = 64
K = 64
BQ = 256
F = DG + NH * NB
NEG = -1e9
THRESH = -1e8


def _feat(x_g, h, scale):
    pieces = [jnp.where(x_g > 0, scale, -scale).astype(jnp.bfloat16)]
    io = jax.lax.broadcasted_iota(jnp.int32, (x_g.shape[0], NB), 1)
    for j in range(NH):
        pieces.append((h[:, j:j + 1] == io).astype(jnp.bfloat16))
    return jnp.concatenate(pieces, axis=1)


def _buckets(x_g, p):
    v = jax.lax.dot_general(x_g, p, (((1,), (0,)), ((), ())),
                            preferred_element_type=jnp.float32)
    return (jnp.floor(v * 0.5).astype(jnp.int32)) & (NB - 1)


def _body(q_ref, k_ref, p0_ref, p1_ref, os_ref, oi_ref,
          kf0_ref, kf1_ref, msk_ref, done_ref):
    i = pl.program_id(1)
    qb = q_ref[0]
    dnt = (((1,), (1,)), ((), ()))

    @pl.when(i == 0)
    def _():
        kb = k_ref[0]
        for g, (p_ref, kf_ref) in enumerate(((p0_ref, kf0_ref),
                                             (p1_ref, kf1_ref))):
            kg = kb[:, g * DG:(g + 1) * DG]
            kh = _buckets(kg, p_ref[...])
            kf_ref[...] = _feat(kg, kh, 1.0)

    vs = []
    for g, (p_ref, kf_ref) in enumerate(((p0_ref, kf0_ref),
                                         (p1_ref, kf1_ref))):
        qg = qb[:, g * DG:(g + 1) * DG]
        qh = _buckets(qg, p_ref[...])
        qf = _feat(qg, qh, 8.0)
        vs.append(jax.lax.dot_general(qf, kf_ref[...], dnt,
                                      preferred_element_type=jnp.float32))

    mask = (vs[0] > 256.5) | (vs[1] > 256.5)
    npos = jnp.sum(mask.astype(jnp.int32))

    os_ref[0] = jnp.full((BQ, K), NEG, jnp.float32)
    oi_ref[0] = jnp.full((BQ, K), -1, jnp.int32)
    done_ref[0] = jnp.where(npos > 0, 0, 1).astype(jnp.int32)

    @pl.when(npos > 0)
    def _():
        scores = jax.lax.dot_general(qb, k_ref[0], dnt,
                                     preferred_element_type=jnp.float32)
        msk_ref[...] = jnp.where(mask, scores, NEG)

    kiota = jax.lax.broadcasted_iota(jnp.int32, (BQ, L), 1)
    liota = jax.lax.broadcasted_iota(jnp.int32, (BQ, K), 1)

    def step(j, carry):
        @pl.when(done_ref[0] == 0)
        def _():
            mm = msk_ref[...]
            m = jnp.max(mm, axis=1)
            bmax = jnp.max(m)

            @pl.when(bmax <= THRESH)
            def _():
                done_ref[0] = 1

            @pl.when(bmax > THRESH)
            def _():
                eq = mm == m[:, None]
                am = jnp.min(jnp.where(eq, kiota, L), axis=1)
                valid = m > THRESH
                sc = jnp.where(valid, m, NEG)
                ix = jnp.where(valid, am, -1)
                os_ref[0] = jnp.where(liota == j, sc[:, None], os_ref[0])
                oi_ref[0] = jnp.where(liota == j, ix[:, None], oi_ref[0])
                pop = (kiota == am[:, None]) & valid[:, None]
                msk_ref[...] = jnp.where(pop, NEG, mm)
        return carry

    jax.lax.fori_loop(0, K, step, 0)


@jax.jit
def _run(q, k, p0, p1):
    out = pl.pallas_call(
        _body,
        grid=(B, L // BQ),
        in_specs=[
            pl.BlockSpec((1, BQ, D), lambda b, i: (b, i, 0)),
            pl.BlockSpec((1, L, D), lambda b, i: (b, 0, 0)),
            pl.BlockSpec((DG, NH), lambda b, i: (0, 0)),
            pl.BlockSpec((DG, NH), lambda b, i: (0, 0)),
        ],
        out_specs=[
            pl.BlockSpec((1, BQ, K), lambda b, i: (b, i, 0)),
            pl.BlockSpec((1, BQ, K), lambda b, i: (b, i, 0)),
        ],
        out_shape=[
            jax.ShapeDtypeStruct((B, L, K), jnp.float32),
            jax.ShapeDtypeStruct((B, L, K), jnp.int32),
        ],
        scratch_shapes=[
            pltpu.VMEM((L, F), jnp.bfloat16),
            pltpu.VMEM((L, F), jnp.bfloat16),
            pltpu.VMEM((BQ, L), jnp.float32),
            pltpu.SMEM((1,), jnp.int32),
        ],
    )(q, k, p0, p1)
    return out[0], out[1]


def kernel(query_up, key_up, lsh_proj_g0, lsh_proj_g1, head_idx=0):
    del head_idx
    return _run(query_up, key_up, lsh_proj_g0, lsh_proj_g1)


# speedup vs baseline: 26.0294x; 1.1264x over previous
import jax
import jax.numpy as jnp
from jax.experimental import pallas as pl
from jax.experimental.pallas import tpu as pltpu

B = 2
L = 2048
D = 64
DG = 32
NH = 4
NB = 64
K = 64
BQ = 256
F = DG + NH * NB
NEG = -1e9
THRESH = -1e8


def _feat(x_g, h, scale):
    n = x_g.shape[0]
    sgn = jnp.where(x_g > 0, scale, -scale).astype(jnp.bfloat16)
    riota = jax.lax.broadcasted_iota(jnp.int32, (NH, NH * NB), 0)
    ciota = jax.lax.broadcasted_iota(jnp.int32, (NH, NH * NB), 1)
    sel = ((ciota // NB) == riota).astype(jnp.float32)
    hb = jax.lax.dot_general(h.astype(jnp.float32), sel,
                             (((1,), (0,)), ((), ())),
                             preferred_element_type=jnp.float32)
    io6 = (jax.lax.broadcasted_iota(jnp.int32, (n, NH * NB), 1)
           & (NB - 1)).astype(jnp.float32)
    onehot = (hb == io6).astype(jnp.bfloat16)
    return jnp.concatenate([sgn, onehot], axis=1)


def _buckets(x_g, p):
    v = jax.lax.dot_general(x_g, p, (((1,), (0,)), ((), ())),
                            preferred_element_type=jnp.float32)
    return (jnp.floor(v * 0.5).astype(jnp.int32)) & (NB - 1)


def _body(q_ref, k_ref, p0_ref, p1_ref, os_ref, oi_ref,
          kf0_ref, kf1_ref, msk_ref, done_ref):
    i = pl.program_id(1)
    qb = q_ref[0]
    dnt = (((1,), (1,)), ((), ()))

    @pl.when(i == 0)
    def _():
        kb = k_ref[0]
        for g, (p_ref, kf_ref) in enumerate(((p0_ref, kf0_ref),
                                             (p1_ref, kf1_ref))):
            kg = kb[:, g * DG:(g + 1) * DG]
            kh = _buckets(kg, p_ref[...])
            kf_ref[...] = _feat(kg, kh, 1.0)

    vs = []
    for g, (p_ref, kf_ref) in enumerate(((p0_ref, kf0_ref),
                                         (p1_ref, kf1_ref))):
        qg = qb[:, g * DG:(g + 1) * DG]
        qh = _buckets(qg, p_ref[...])
        qf = _feat(qg, qh, 8.0)
        vs.append(jax.lax.dot_general(qf, kf_ref[...], dnt,
                                      preferred_element_type=jnp.float32))

    mask = (vs[0] > 256.5) | (vs[1] > 256.5)
    npos = jnp.sum(mask.astype(jnp.int32))

    os_ref[0] = jnp.full((BQ, K), NEG, jnp.float32)
    oi_ref[0] = jnp.full((BQ, K), -1, jnp.int32)
    done_ref[0] = jnp.where(npos > 0, 0, 1).astype(jnp.int32)

    @pl.when(npos > 0)
    def _():
        scores = jax.lax.dot_general(qb, k_ref[0], dnt,
                                     preferred_element_type=jnp.float32)
        msk_ref[...] = jnp.where(mask, scores, NEG)

    kiota = jax.lax.broadcasted_iota(jnp.int32, (BQ, L), 1)
    liota = jax.lax.broadcasted_iota(jnp.int32, (BQ, K), 1)

    def step(j, carry):
        @pl.when(done_ref[0] == 0)
        def _():
            mm = msk_ref[...]
            m = jnp.max(mm, axis=1)
            bmax = jnp.max(m)

            @pl.when(bmax <= THRESH)
            def _():
                done_ref[0] = 1

            @pl.when(bmax > THRESH)
            def _():
                eq = mm == m[:, None]
                am = jnp.min(jnp.where(eq, kiota, L), axis=1)
                valid = m > THRESH
                sc = jnp.where(valid, m, NEG)
                ix = jnp.where(valid, am, -1)
                os_ref[0] = jnp.where(liota == j, sc[:, None], os_ref[0])
                oi_ref[0] = jnp.where(liota == j, ix[:, None], oi_ref[0])
                pop = (kiota == am[:, None]) & valid[:, None]
                msk_ref[...] = jnp.where(pop, NEG, mm)
        return carry

    jax.lax.fori_loop(0, K, step, 0)


@jax.jit
def _run(q, k, p0, p1):
    out = pl.pallas_call(
        _body,
        grid=(B, L // BQ),
        in_specs=[
            pl.BlockSpec((1, BQ, D), lambda b, i: (b, i, 0)),
            pl.BlockSpec((1, L, D), lambda b, i: (b, 0, 0)),
            pl.BlockSpec((DG, NH), lambda b, i: (0, 0)),
            pl.BlockSpec((DG, NH), lambda b, i: (0, 0)),
        ],
        out_specs=[
            pl.BlockSpec((1, BQ, K), lambda b, i: (b, i, 0)),
            pl.BlockSpec((1, BQ, K), lambda b, i: (b, i, 0)),
        ],
        out_shape=[
            jax.ShapeDtypeStruct((B, L, K), jnp.float32),
            jax.ShapeDtypeStruct((B, L, K), jnp.int32),
        ],
        scratch_shapes=[
            pltpu.VMEM((L, F), jnp.bfloat16),
            pltpu.VMEM((L, F), jnp.bfloat16),
            pltpu.VMEM((BQ, L), jnp.float32),
            pltpu.SMEM((1,), jnp.int32),
        ],
    )(q, k, p0, p1)
    return out[0], out[1]


def kernel(query_up, key_up, lsh_proj_g0, lsh_proj_g1, head_idx=0):
    del head_idx
    return _run(query_up, key_up, lsh_proj_g0, lsh_proj_g1)
